# baseline (speedup 1.0000x reference)
import jax
import jax.numpy as jnp
from jax.experimental import pallas as pl
from jax.experimental.pallas import tpu as pltpu

_B = 1024
_DIN = 64
_DF = 16
_DA = _DF + 2
_K = 100000
_TK = 2048
_NK = (_K + _TK - 1) // _TK
_RB = 128
_W = 128
_BIG = 1e9


def _knn_body(x_ref, w_ref, b_ref, t_ref, o_ref, ta_ref, m1_ref, m2_ref):
    k = pl.program_id(0)

    @pl.when(k == 0)
    def _init():
        test = jnp.dot(x_ref[...], w_ref[...],
                       preferred_element_type=jnp.float32) + b_ref[...]
        q2 = jnp.sum(test * test, axis=1, keepdims=True)
        ta_ref[...] = jnp.concatenate(
            [-2.0 * test, jnp.ones((_B, 1), jnp.float32), q2], axis=1)
        m1_ref[...] = jnp.full((_B, _W), jnp.inf, jnp.bfloat16)
        m2_ref[...] = jnp.full((_B, _W), jnp.inf, jnp.bfloat16)

    row = jax.lax.broadcasted_iota(jnp.int32, (_TK, 1), 0)
    ok = row + k * _TK < _K
    tt = jnp.where(ok, t_ref[...], 0.0)
    t2 = jnp.sum(tt * tt, axis=1, keepdims=True)
    t2 = jnp.where(ok, t2, _BIG)
    taug = jnp.concatenate(
        [tt, t2, jnp.ones((_TK, 1), jnp.float32)], axis=1)
    s = jax.lax.dot_general(
        ta_ref[...], taug, (((1,), (1,)), ((), ())),
        preferred_element_type=jnp.float32).astype(jnp.bfloat16)

    for rb in range(_B // _RB):
        r = pl.ds(rb * _RB, _RB)
        lo = m1_ref[r, :]
        hi = m2_ref[r, :]
        for c in range(_TK // _W):
            v = s[rb * _RB:(rb + 1) * _RB, c * _W:(c + 1) * _W]
            nlo = jnp.minimum(lo, v)
            hi = jnp.minimum(hi, jnp.maximum(lo, v))
            lo = nlo
        m1_ref[r, :] = lo
        m2_ref[r, :] = hi

    @pl.when(k == _NK - 1)
    def _fin():
        lo = m1_ref[...]
        hi = m2_ref[...]
        w = _W
        while w > 1:
            w //= 2
            la, lb = lo[:, :w], lo[:, w:]
            ha, hb = hi[:, :w], hi[:, w:]
            lo = jnp.minimum(la, lb)
            hi = jnp.minimum(jnp.maximum(la, lb), jnp.minimum(ha, hb))
        o_ref[...] = lo.astype(jnp.float32) + hi.astype(jnp.float32)


def kernel(input, W, b, train_features):
    out = pl.pallas_call(
        _knn_body,
        grid=(_NK,),
        in_specs=[
            pl.BlockSpec((_B, _DIN), lambda k: (0, 0)),
            pl.BlockSpec((_DIN, _DF), lambda k: (0, 0)),
            pl.BlockSpec((1, _DF), lambda k: (0, 0)),
            pl.BlockSpec((_TK, _DF), lambda k: (k, 0)),
        ],
        out_specs=pl.BlockSpec((_B, 1), lambda k: (0, 0)),
        out_shape=jax.ShapeDtypeStruct((_B, 1), jnp.float32),
        scratch_shapes=[
            pltpu.VMEM((_B, _DA), jnp.float32),
            pltpu.VMEM((_B, _W), jnp.bfloat16),
            pltpu.VMEM((_B, _W), jnp.bfloat16),
        ],
    )(input, W, b.reshape(1, _DF), train_features)
    return out.reshape(_B)


# speedup vs baseline: 1.1281x; 1.1281x over previous
import jax
import jax.numpy as jnp
from jax.experimental import pallas as pl
from jax.experimental.pallas import tpu as pltpu

_B = 1024
_DIN = 64
_DF = 16
_K = 100000
_TK = 4096
_TK2 = 2 * _TK
_NG = (_K + _TK2 - 1) // _TK2
_RB = 128
_W = 128
_MC = _TK // 4
_BIG = 1e9


def _taug(tt_raw, base):
    row = jax.lax.broadcasted_iota(jnp.int32, (_TK, 1), 0)
    ok = row + base < _K
    tt = jnp.where(ok, tt_raw, 0.0)
    t2 = jnp.sum(tt * tt, axis=1, keepdims=True)
    t2 = jnp.where(ok, t2, _BIG)
    return jnp.concatenate([tt, t2], axis=1)


def _half(ta, taug, read_ref, write_ref, m1_ref, m2_ref):
    for rb in range(_B // _RB):
        if rb % 2 == 0:
            c0 = (rb // 2) * _MC
            write_ref[:, pl.ds(c0, _MC)] = jax.lax.dot_general(
                ta, taug[c0:c0 + _MC, :], (((1,), (1,)), ((), ())),
                preferred_element_type=jnp.float32)
        r = pl.ds(rb * _RB, _RB)
        lo = m1_ref[r, :]
        hi = m2_ref[r, :]
        for c in range(_TK // _W):
            v = read_ref[r, pl.ds(c * _W, _W)]
            nlo = jnp.minimum(lo, v)
            hi = jnp.minimum(hi, jnp.maximum(lo, v))
            lo = nlo
        m1_ref[r, :] = lo
        m2_ref[r, :] = hi


def _knn_body(x_ref, w_ref, b_ref, t_ref, o_ref, ta_ref, m1_ref, m2_ref,
              sa_ref, sb_ref):
    g = pl.program_id(0)

    @pl.when(g == 0)
    def _init():
        test = jnp.dot(x_ref[...], w_ref[...],
                       preferred_element_type=jnp.float32) + b_ref[...]
        ta_ref[...] = jnp.concatenate(
            [-2.0 * test, jnp.ones((_B, 1), jnp.float32)], axis=1)
        m1_ref[...] = jnp.full((_B, _W), jnp.inf, jnp.float32)
        m2_ref[...] = jnp.full((_B, _W), jnp.inf, jnp.float32)
        sb_ref[...] = jnp.full((_B, _TK), jnp.inf, jnp.float32)

    ta = ta_ref[...]
    base = g * _TK2

    taug_a = _taug(t_ref[0:_TK, :], base)
    _half(ta, taug_a, sb_ref, sa_ref, m1_ref, m2_ref)
    taug_b = _taug(t_ref[_TK:_TK2, :], base + _TK)
    _half(ta, taug_b, sa_ref, sb_ref, m1_ref, m2_ref)

    @pl.when(g == _NG - 1)
    def _fin():
        lo = m1_ref[...]
        hi = m2_ref[...]
        w = _W
        while w > 1:
            w //= 2
            la, lb = lo[:, :w], lo[:, w:]
            ha, hb = hi[:, :w], hi[:, w:]
            lo = jnp.minimum(la, lb)
            hi = jnp.minimum(jnp.maximum(la, lb), jnp.minimum(ha, hb))
        q2 = 0.25 * jnp.sum(ta[:, :_DF] * ta[:, :_DF], axis=1, keepdims=True)
        o_ref[...] = lo + hi + 2.0 * q2


def kernel(input, W, b, train_features):
    out = pl.pallas_call(
        _knn_body,
        grid=(_NG,),
        in_specs=[
            pl.BlockSpec((_B, _DIN), lambda g: (0, 0)),
            pl.BlockSpec((_DIN, _DF), lambda g: (0, 0)),
            pl.BlockSpec((1, _DF), lambda g: (0, 0)),
            pl.BlockSpec((_TK2, _DF), lambda g: (g, 0)),
        ],
        out_specs=pl.BlockSpec((_B, 1), lambda g: (0, 0)),
        out_shape=jax.ShapeDtypeStruct((_B, 1), jnp.float32),
        scratch_shapes=[
            pltpu.VMEM((_B, _DF + 1), jnp.float32),
            pltpu.VMEM((_B, _W), jnp.float32),
            pltpu.VMEM((_B, _W), jnp.float32),
            pltpu.VMEM((_B, _TK), jnp.float32),
            pltpu.VMEM((_B, _TK), jnp.float32),
        ],
    )(input, W, b.reshape(1, _DF), train_features)
    return out.reshape(_B)


# speedup vs baseline: 1.1469x; 1.0167x over previous
import jax
import jax.numpy as jnp
from jax.experimental import pallas as pl
from jax.experimental.pallas import tpu as pltpu

_B = 1024
_DIN = 64
_DF = 16
_K = 100000
_TK = 2048
_TK2 = 2 * _TK
_NG = (_K + _TK2 - 1) // _TK2
_RB = 128
_W = 128
_MC = _TK // 4
_BIG = 1e9


def _taug(tt_raw, base):
    row = jax.lax.broadcasted_iota(jnp.int32, (_TK, 1), 0)
    ok = row + base < _K
    tt = jnp.where(ok, tt_raw, 0.0)
    t2 = jnp.sum(tt * tt, axis=1, keepdims=True)
    t2 = jnp.where(ok, t2, _BIG)
    return jnp.concatenate([tt, t2], axis=1)


def _half(ta, taug, read_ref, write_ref, m1_ref, m2_ref):
    for rb in range(_B // _RB):
        if rb % 2 == 0:
            c0 = (rb // 2) * _MC
            write_ref[:, pl.ds(c0, _MC)] = jax.lax.dot_general(
                ta, taug[c0:c0 + _MC, :], (((1,), (1,)), ((), ())),
                preferred_element_type=jnp.float32)
        r = pl.ds(rb * _RB, _RB)
        lo = m1_ref[r, :]
        hi = m2_ref[r, :]
        for c in range(_TK // _W):
            v = read_ref[r, pl.ds(c * _W, _W)]
            nlo = jnp.minimum(lo, v)
            hi = jnp.minimum(hi, jnp.maximum(lo, v))
            lo = nlo
        m1_ref[r, :] = lo
        m2_ref[r, :] = hi


def _knn_body(x_ref, w_ref, b_ref, t_ref, o_ref, ta_ref, m1_ref, m2_ref,
              sa_ref, sb_ref):
    g = pl.program_id(0)

    @pl.when(g == 0)
    def _init():
        test = jnp.dot(x_ref[...], w_ref[...],
                       preferred_element_type=jnp.float32) + b_ref[...]
        ta_ref[...] = jnp.concatenate(
            [-2.0 * test, jnp.ones((_B, 1), jnp.float32)], axis=1)
        m1_ref[...] = jnp.full((_B, _W), jnp.inf, jnp.float32)
        m2_ref[...] = jnp.full((_B, _W), jnp.inf, jnp.float32)
        sb_ref[...] = jnp.full((_B, _TK), jnp.inf, jnp.float32)

    ta = ta_ref[...]
    base = g * _TK2

    taug_a = _taug(t_ref[0:_TK, :], base)
    _half(ta, taug_a, sb_ref, sa_ref, m1_ref, m2_ref)
    taug_b = _taug(t_ref[_TK:_TK2, :], base + _TK)
    _half(ta, taug_b, sa_ref, sb_ref, m1_ref, m2_ref)

    @pl.when(g == _NG - 1)
    def _fin():
        lo = m1_ref[...]
        hi = m2_ref[...]
        w = _W
        while w > 1:
            w //= 2
            la, lb = lo[:, :w], lo[:, w:]
            ha, hb = hi[:, :w], hi[:, w:]
            lo = jnp.minimum(la, lb)
            hi = jnp.minimum(jnp.maximum(la, lb), jnp.minimum(ha, hb))
        q2 = 0.25 * jnp.sum(ta[:, :_DF] * ta[:, :_DF], axis=1, keepdims=True)
        o_ref[...] = lo + hi + 2.0 * q2


def kernel(input, W, b, train_features):
    out = pl.pallas_call(
        _knn_body,
        grid=(_NG,),
        in_specs=[
            pl.BlockSpec((_B, _DIN), lambda g: (0, 0)),
            pl.BlockSpec((_DIN, _DF), lambda g: (0, 0)),
            pl.BlockSpec((1, _DF), lambda g: (0, 0)),
            pl.BlockSpec((_TK2, _DF), lambda g: (g, 0)),
        ],
        out_specs=pl.BlockSpec((_B, 1), lambda g: (0, 0)),
        out_shape=jax.ShapeDtypeStruct((_B, 1), jnp.float32),
        scratch_shapes=[
            pltpu.VMEM((_B, _DF + 1), jnp.float32),
            pltpu.VMEM((_B, _W), jnp.float32),
            pltpu.VMEM((_B, _W), jnp.float32),
            pltpu.VMEM((_B, _TK), jnp.float32),
            pltpu.VMEM((_B, _TK), jnp.float32),
        ],
    )(input, W, b.reshape(1, _DF), train_features)
    return out.reshape(_B)
